# Initial kernel scaffold; baseline (speedup 1.0000x reference)
import jax
import jax.numpy as jnp
from jax.experimental import pallas as pl


def kernel(features, edge_index, W0, W1, W2):
    raise NotImplementedError("write your pallas kernel here")


# speedup vs baseline: 3.9691x; 3.9691x over previous
import functools

import jax
import jax.numpy as jnp
from jax import lax
from jax.experimental import pallas as pl
from jax.experimental.pallas import tpu as pltpu
from jax.experimental.pallas import tpu_sc as plsc

_N = 10000
_E = 320000
_NC = 2
_NS = 16
_EPT = _E // _NS
_CHUNK = 80
_NCHUNK = _EPT // _CHUNK
_RPT = _N // _NS
_ZR = 125
_R = 2000


def _mm_first_body(x_ref, w_ref, o_ref):
    h = jnp.dot(x_ref[...], w_ref[...], preferred_element_type=jnp.float32)
    o_ref[0, :, :] = h[:, :128]
    o_ref[1, :, :] = h[:, 128:]


def _mm_first(x, w):
    return pl.pallas_call(
        _mm_first_body,
        grid=(_N // _R,),
        in_specs=[pl.BlockSpec((_R, 128), lambda i: (i, 0)),
                  pl.BlockSpec((128, 256), lambda i: (0, 0))],
        out_specs=pl.BlockSpec((2, _R, 128), lambda i: (0, i, 0)),
        out_shape=jax.ShapeDtypeStruct((2, _N, 128), jnp.float32),
    )(x, w)


def _mm_mid_body(a_ref, w_ref, o_ref, *, pw_out):
    x = jnp.concatenate([a_ref[0, :, :], a_ref[1, :, :]], axis=1)
    h = jnp.dot(jnp.tanh(x), w_ref[...], preferred_element_type=jnp.float32)
    o_ref[0, :, :] = h[:, :pw_out]
    o_ref[1, :, :] = h[:, pw_out:]


def _mm_mid(a, w, pw_out):
    return pl.pallas_call(
        functools.partial(_mm_mid_body, pw_out=pw_out),
        grid=(_N // _R,),
        in_specs=[pl.BlockSpec((2, _R, 128), lambda i: (0, i, 0)),
                  pl.BlockSpec(w.shape, lambda i: (0, 0))],
        out_specs=pl.BlockSpec((2, _R, pw_out), lambda i: (0, i, 0)),
        out_shape=jax.ShapeDtypeStruct((2, _N, pw_out), jnp.float32),
    )(a, w)


def _mm_full_body(a_ref, w_ref, o_ref):
    x = jnp.concatenate([a_ref[0, :, :], a_ref[1, :, :]], axis=1)
    o_ref[...] = jnp.dot(jnp.tanh(x), w_ref[...],
                         preferred_element_type=jnp.float32)


def _mm_full(a, w):
    dout = w.shape[1]
    return pl.pallas_call(
        _mm_full_body,
        grid=(_N // _R,),
        in_specs=[pl.BlockSpec((2, _R, 128), lambda i: (0, i, 0)),
                  pl.BlockSpec(w.shape, lambda i: (0, 0))],
        out_specs=pl.BlockSpec((_R, dout), lambda i: (i, 0)),
        out_shape=jax.ShapeDtypeStruct((_N, dout), jnp.float32),
    )(a, w)


def _tanh_sum_body(a_ref, o_ref):
    o_ref[...] = jnp.tanh(a_ref[0, :, :] + a_ref[1, :, :])


def _final_tanh_sum(a):
    return pl.pallas_call(
        _tanh_sum_body,
        grid=(_N // _R,),
        in_specs=[pl.BlockSpec((2, _R, 128), lambda i: (0, i, 0))],
        out_specs=pl.BlockSpec((_R, 128), lambda i: (i, 0)),
        out_shape=jax.ShapeDtypeStruct((_N, 128), jnp.float32),
    )(a)


@functools.lru_cache(maxsize=None)
def _make_agg(pw):
    mesh = plsc.VectorSubcoreMesh(core_axis_name="c", subcore_axis_name="s")

    @functools.partial(
        pl.kernel, mesh=mesh,
        out_type=jax.ShapeDtypeStruct((_NC * _N, pw), jnp.float32),
        scratch_types=[
            pltpu.VMEM((_CHUNK,), jnp.int32),
            pltpu.VMEM((_CHUNK,), jnp.int32),
            pltpu.VMEM((_CHUNK, pw), jnp.float32),
            pltpu.VMEM((_ZR, pw), jnp.float32),
            pltpu.VMEM_SHARED((_N, pw), jnp.float32),
            pltpu.SemaphoreType.DMA,
        ],
    )
    def agg(h_hbm, src_hbm, dst_hbm, out_hbm, sidx, didx, rows, zbuf, accum,
            sem):
        cid = lax.axis_index("c")
        sid = lax.axis_index("s")

        def zrow(r, _):
            def zcol(j, _):
                zbuf[r, pl.ds(j * 16, 16)] = jnp.zeros((16,), jnp.float32)
                return 0
            return lax.fori_loop(0, pw // 16, zcol, 0)
        lax.fori_loop(0, _ZR, zrow, 0)
        rbase = sid * _RPT
        for z in range(_RPT // _ZR):
            pltpu.sync_copy(zbuf, accum.at[pl.ds(rbase + z * _ZR, _ZR)])
        plsc.subcore_barrier()

        ebase = sid * _EPT
        off0 = cid * _N

        def chunk(i, _):
            off = ebase + i * _CHUNK
            pltpu.sync_copy(src_hbm.at[pl.ds(off, _CHUNK)], sidx)
            pltpu.sync_copy(dst_hbm.at[pl.ds(off, _CHUNK)], didx)

            def addoff(j, _):
                sl = pl.ds(j * 16, 16)
                sidx[sl] = sidx[sl] + off0
                return 0
            lax.fori_loop(0, _CHUNK // 16, addoff, 0)
            pltpu.async_copy(h_hbm.at[sidx], rows, sem).wait()
            pltpu.sync_copy(rows, accum.at[didx], add=True)
            return 0
        lax.fori_loop(0, _NCHUNK, chunk, 0)
        plsc.subcore_barrier()

        cbase = sid * 624
        pltpu.sync_copy(accum.at[pl.ds(cbase, 624)],
                        out_hbm.at[pl.ds(off0 + cbase, 624)])

        @pl.when(sid == _NS - 1)
        def _tail():
            pltpu.sync_copy(accum.at[pl.ds(9984, 16)],
                            out_hbm.at[pl.ds(off0 + 9984, 16)])

    return agg


@functools.lru_cache(maxsize=None)
def _make_agg_esplit():
    epc = _E // _NC
    ept = epc // _NS
    nchunk = ept // _CHUNK
    mesh = plsc.VectorSubcoreMesh(core_axis_name="c", subcore_axis_name="s")

    @functools.partial(
        pl.kernel, mesh=mesh,
        out_type=jax.ShapeDtypeStruct((_NC * _N, 128), jnp.float32),
        scratch_types=[
            pltpu.VMEM((_CHUNK,), jnp.int32),
            pltpu.VMEM((_CHUNK,), jnp.int32),
            pltpu.VMEM((_CHUNK, 128), jnp.float32),
            pltpu.VMEM((_ZR, 128), jnp.float32),
            pltpu.VMEM_SHARED((_N, 128), jnp.float32),
            pltpu.SemaphoreType.DMA,
        ],
    )
    def agg(h_hbm, src_hbm, dst_hbm, out_hbm, sidx, didx, rows, zbuf, accum,
            sem):
        cid = lax.axis_index("c")
        sid = lax.axis_index("s")

        def zrow(r, _):
            def zcol(j, _):
                zbuf[r, pl.ds(j * 16, 16)] = jnp.zeros((16,), jnp.float32)
                return 0
            return lax.fori_loop(0, 128 // 16, zcol, 0)
        lax.fori_loop(0, _ZR, zrow, 0)
        rbase = sid * _RPT
        for z in range(_RPT // _ZR):
            pltpu.sync_copy(zbuf, accum.at[pl.ds(rbase + z * _ZR, _ZR)])
        plsc.subcore_barrier()

        ebase = cid * epc + sid * ept
        off0 = cid * _N

        def chunk(i, _):
            off = ebase + i * _CHUNK
            pltpu.sync_copy(src_hbm.at[pl.ds(off, _CHUNK)], sidx)
            pltpu.sync_copy(dst_hbm.at[pl.ds(off, _CHUNK)], didx)
            pltpu.async_copy(h_hbm.at[sidx], rows, sem).wait()
            pltpu.sync_copy(rows, accum.at[didx], add=True)
            return 0
        lax.fori_loop(0, nchunk, chunk, 0)
        plsc.subcore_barrier()

        cbase = sid * 624
        pltpu.sync_copy(accum.at[pl.ds(cbase, 624)],
                        out_hbm.at[pl.ds(off0 + cbase, 624)])

        @pl.when(sid == _NS - 1)
        def _tail():
            pltpu.sync_copy(accum.at[pl.ds(9984, 16)],
                            out_hbm.at[pl.ds(off0 + 9984, 16)])

    return agg


def kernel(features, edge_index, W0, W1, W2):
    src = edge_index[0].astype(jnp.int32)
    dst = edge_index[1].astype(jnp.int32)
    agg128 = _make_agg(128)
    h0 = _mm_first(features, W0).reshape(_NC * _N, 128)
    a0 = agg128(h0, src, dst).reshape(_NC, _N, 128)
    h1 = _mm_mid(a0, W1, 128).reshape(_NC * _N, 128)
    a1 = agg128(h1, src, dst).reshape(_NC, _N, 128)
    h2 = _mm_full(a1, W2)
    a2 = _make_agg_esplit()(h2, src, dst).reshape(_NC, _N, 128)
    return _final_tanh_sum(a2)


# speedup vs baseline: 9.8155x; 2.4730x over previous
import functools

import jax
import jax.numpy as jnp
from jax import lax
from jax.experimental import pallas as pl
from jax.experimental.pallas import tpu as pltpu
from jax.experimental.pallas import tpu_sc as plsc

_N = 10000
_E = 320000
_NC = 2
_NS = 16
_CHUNK = 80
_EBLK = 10000
_NCHUNK = _EBLK // _CHUNK
_RPT = _N // _NS
_ZR = 25
_R = 2000


def _mm_first_body(x_ref, w_ref, o_ref):
    h = jnp.dot(x_ref[...], w_ref[...], preferred_element_type=jnp.float32)
    o_ref[0, :, :] = h[:, :128]
    o_ref[1, :, :] = h[:, 128:]


def _mm_first(x, w):
    return pl.pallas_call(
        _mm_first_body,
        grid=(_N // _R,),
        in_specs=[pl.BlockSpec((_R, 128), lambda i: (i, 0)),
                  pl.BlockSpec((128, 256), lambda i: (0, 0))],
        out_specs=pl.BlockSpec((2, _R, 128), lambda i: (0, i, 0)),
        out_shape=jax.ShapeDtypeStruct((2, _N, 128), jnp.float32),
    )(x, w)


def _mm_mid_body(a_ref, w_ref, o_ref):
    x = jnp.concatenate([a_ref[0, :, :], a_ref[1, :, :]], axis=1)
    h = jnp.dot(jnp.tanh(x), w_ref[...], preferred_element_type=jnp.float32)
    o_ref[0, :, :] = h[:, :128]
    o_ref[1, :, :] = h[:, 128:]


def _mm_mid(a, w):
    return pl.pallas_call(
        _mm_mid_body,
        grid=(_N // _R,),
        in_specs=[pl.BlockSpec((2, _R, 128), lambda i: (0, i, 0)),
                  pl.BlockSpec((256, 256), lambda i: (0, 0))],
        out_specs=pl.BlockSpec((2, _R, 128), lambda i: (0, i, 0)),
        out_shape=jax.ShapeDtypeStruct((2, _N, 128), jnp.float32),
    )(a, w)


def _mm_full_body(a_ref, w_ref, o_ref):
    x = jnp.concatenate([a_ref[0, :, :], a_ref[1, :, :]], axis=1)
    o_ref[...] = jnp.dot(jnp.tanh(x), w_ref[...],
                         preferred_element_type=jnp.float32)


def _mm_full(a, w):
    dout = w.shape[1]
    return pl.pallas_call(
        _mm_full_body,
        grid=(_N // _R,),
        in_specs=[pl.BlockSpec((2, _R, 128), lambda i: (0, i, 0)),
                  pl.BlockSpec(w.shape, lambda i: (0, 0))],
        out_specs=pl.BlockSpec((_R, dout), lambda i: (i, 0)),
        out_shape=jax.ShapeDtypeStruct((_N, dout), jnp.float32),
    )(a, w)


def _tanh_sum_body(a_ref, o_ref):
    o_ref[...] = jnp.tanh(a_ref[0, :, :] + a_ref[1, :, :])


def _final_tanh_sum(a):
    return pl.pallas_call(
        _tanh_sum_body,
        grid=(_N // _R,),
        in_specs=[pl.BlockSpec((2, _R, 128), lambda i: (0, i, 0))],
        out_specs=pl.BlockSpec((_R, 128), lambda i: (i, 0)),
        out_shape=jax.ShapeDtypeStruct((_N, 128), jnp.float32),
    )(a)


@functools.lru_cache(maxsize=None)
def _make_agg(edge_split):
    nblk = 1 if edge_split else 2
    mesh = plsc.VectorSubcoreMesh(core_axis_name="c", subcore_axis_name="s")

    @functools.partial(
        pl.kernel, mesh=mesh,
        out_type=jax.ShapeDtypeStruct((_NC * _N, 128), jnp.float32),
        scratch_types=[
            pltpu.VMEM((_EBLK,), jnp.int32),
            pltpu.VMEM((_EBLK,), jnp.int32),
            pltpu.VMEM((_CHUNK, 128), jnp.float32),
            pltpu.VMEM((_CHUNK, 128), jnp.float32),
            pltpu.VMEM((_ZR, 128), jnp.float32),
            pltpu.VMEM_SHARED((_N, 128), jnp.float32),
            pltpu.SemaphoreType.DMA,
            pltpu.SemaphoreType.DMA,
        ],
    )
    def agg(h0_hbm, h1_hbm, src_hbm, dst_hbm, out_hbm, sidx, didx, rows_a,
            rows_b, zbuf, accum, sem_a, sem_b):
        cid = lax.axis_index("c")
        sid = lax.axis_index("s")

        def zrow(r, _):
            def zcol(j, _):
                zbuf[r, pl.ds(j * 16, 16)] = jnp.zeros((16,), jnp.float32)
                return 0
            return lax.fori_loop(0, 128 // 16, zcol, 0)
        lax.fori_loop(0, _ZR, zrow, 0)
        rbase = sid * _RPT
        for z in range(_RPT // _ZR):
            pltpu.sync_copy(zbuf, accum.at[pl.ds(rbase + z * _ZR, _ZR)])
        plsc.subcore_barrier()

        off0 = cid * _N

        def gather(i, buf, sem):
            idx = sidx.at[pl.ds(i * _CHUNK, _CHUNK)]

            @pl.when(cid == 0)
            def _g0():
                pltpu.make_async_copy(h0_hbm.at[idx], buf, sem).start()

            @pl.when(cid == 1)
            def _g1():
                pltpu.make_async_copy(h1_hbm.at[idx], buf, sem).start()

        def wait_gather(i, buf, sem):
            idx = sidx.at[pl.ds(i * _CHUNK, _CHUNK)]
            pltpu.make_async_copy(h0_hbm.at[idx], buf, sem).wait()

        def scatter(i, buf):
            idx = didx.at[pl.ds(i * _CHUNK, _CHUNK)]
            pltpu.sync_copy(buf, accum.at[idx], add=True)

        for blk in range(nblk):
            if edge_split:
                ebase = cid * (_E // _NC) + sid * _EBLK
            else:
                ebase = sid * (nblk * _EBLK) + blk * _EBLK
            pltpu.sync_copy(src_hbm.at[pl.ds(ebase, _EBLK)], sidx)
            pltpu.sync_copy(dst_hbm.at[pl.ds(ebase, _EBLK)], didx)

            gather(0, rows_a, sem_a)

            def pair(p, _):
                i = p * 2
                gather(i + 1, rows_b, sem_b)
                wait_gather(i, rows_a, sem_a)
                scatter(i, rows_a)

                @pl.when(i + 2 < _NCHUNK)
                def _next():
                    gather(i + 2, rows_a, sem_a)
                wait_gather(i + 1, rows_b, sem_b)
                scatter(i + 1, rows_b)
                return 0
            lax.fori_loop(0, _NCHUNK // 2, pair, 0)
            wait_gather(_NCHUNK - 1, rows_a, sem_a)
            scatter(_NCHUNK - 1, rows_a)
        plsc.subcore_barrier()

        cbase = sid * 624
        pltpu.sync_copy(accum.at[pl.ds(cbase, 624)],
                        out_hbm.at[pl.ds(off0 + cbase, 624)])

        @pl.when(sid == _NS - 1)
        def _tail():
            pltpu.sync_copy(accum.at[pl.ds(9984, 16)],
                            out_hbm.at[pl.ds(off0 + 9984, 16)])

    return agg


def kernel(features, edge_index, W0, W1, W2):
    src = edge_index[0].astype(jnp.int32)
    dst = edge_index[1].astype(jnp.int32)
    fagg = _make_agg(False)
    eagg = _make_agg(True)
    hp = _mm_first(features, W0)
    a0 = fagg(hp[0], hp[1], src, dst).reshape(_NC, _N, 128)
    hp = _mm_mid(a0, W1)
    a1 = fagg(hp[0], hp[1], src, dst).reshape(_NC, _N, 128)
    h2 = _mm_full(a1, W2)
    a2 = eagg(h2, h2, src, dst).reshape(_NC, _N, 128)
    return _final_tanh_sum(a2)


# speedup vs baseline: 10.2674x; 1.0460x over previous
import functools

import jax
import jax.numpy as jnp
from jax import lax
from jax.experimental import pallas as pl
from jax.experimental.pallas import tpu as pltpu
from jax.experimental.pallas import tpu_sc as plsc

_N = 10000
_E = 320000
_NC = 2
_NS = 16
_CHUNK = 80
_EBLK = 10000
_NCHUNK = _EBLK // _CHUNK
_RPT = _N // _NS
_ZR = 25
_R = 2000


def _mm_first_body(x_ref, w_ref, o_ref):
    h = jnp.dot(x_ref[...], w_ref[...], preferred_element_type=jnp.float32)
    o_ref[0, :, :] = h[:, :128]
    o_ref[1, :, :] = h[:, 128:]


def _mm_first(x, w):
    return pl.pallas_call(
        _mm_first_body,
        grid=(_N // _R,),
        in_specs=[pl.BlockSpec((_R, 128), lambda i: (i, 0)),
                  pl.BlockSpec((128, 256), lambda i: (0, 0))],
        out_specs=pl.BlockSpec((2, _R, 128), lambda i: (0, i, 0)),
        out_shape=jax.ShapeDtypeStruct((2, _N, 128), jnp.float32),
    )(x, w)


def _mm_mid_body(a_ref, w_ref, o_ref):
    x = jnp.concatenate([a_ref[0, :, :], a_ref[1, :, :]], axis=1)
    h = jnp.dot(jnp.tanh(x), w_ref[...], preferred_element_type=jnp.float32)
    o_ref[0, :, :] = h[:, :128]
    o_ref[1, :, :] = h[:, 128:]


def _mm_mid(a, w):
    return pl.pallas_call(
        _mm_mid_body,
        grid=(_N // _R,),
        in_specs=[pl.BlockSpec((2, _R, 128), lambda i: (0, i, 0)),
                  pl.BlockSpec((256, 256), lambda i: (0, 0))],
        out_specs=pl.BlockSpec((2, _R, 128), lambda i: (0, i, 0)),
        out_shape=jax.ShapeDtypeStruct((2, _N, 128), jnp.float32),
    )(a, w)


def _mm_full_body(a_ref, w_ref, o_ref):
    x = jnp.concatenate([a_ref[0, :, :], a_ref[1, :, :]], axis=1)
    o_ref[...] = jnp.dot(jnp.tanh(x), w_ref[...],
                         preferred_element_type=jnp.float32)


def _mm_full(a, w):
    dout = w.shape[1]
    return pl.pallas_call(
        _mm_full_body,
        grid=(_N // _R,),
        in_specs=[pl.BlockSpec((2, _R, 128), lambda i: (0, i, 0)),
                  pl.BlockSpec(w.shape, lambda i: (0, 0))],
        out_specs=pl.BlockSpec((_R, dout), lambda i: (i, 0)),
        out_shape=jax.ShapeDtypeStruct((_N, dout), jnp.float32),
    )(a, w)


def _tanh_sum_body(a_ref, o_ref):
    o_ref[...] = jnp.tanh(a_ref[0, :, :] + a_ref[1, :, :])


def _final_tanh_sum(a):
    return pl.pallas_call(
        _tanh_sum_body,
        grid=(_N // _R,),
        in_specs=[pl.BlockSpec((2, _R, 128), lambda i: (0, i, 0))],
        out_specs=pl.BlockSpec((_R, 128), lambda i: (i, 0)),
        out_shape=jax.ShapeDtypeStruct((_N, 128), jnp.float32),
    )(a)


@functools.lru_cache(maxsize=None)
def _make_agg(edge_split):
    chunk = 80 if edge_split else 160
    eblk = 25 * chunk
    ept = (_E // (_NC * _NS)) if edge_split else (_E // _NS)
    nblk = ept // eblk
    mesh = plsc.VectorSubcoreMesh(core_axis_name="c", subcore_axis_name="s")

    @functools.partial(
        pl.kernel, mesh=mesh,
        out_type=jax.ShapeDtypeStruct((_NC * _N, 128), jnp.float32),
        scratch_types=[
            pltpu.VMEM((eblk,), jnp.int32),
            pltpu.VMEM((eblk,), jnp.int32),
            pltpu.VMEM((chunk, 128), jnp.float32),
            pltpu.VMEM((chunk, 128), jnp.float32),
            pltpu.VMEM_SHARED((_N, 128), jnp.float32),
            pltpu.SemaphoreType.DMA,
            pltpu.SemaphoreType.DMA,
        ],
    )
    def agg(h0_hbm, h1_hbm, src_hbm, dst_hbm, out_hbm, sidx, didx, rows_a,
            rows_b, accum, sem_a, sem_b):
        cid = lax.axis_index("c")
        sid = lax.axis_index("s")

        def zrow(r, _):
            def zcol(j, _):
                rows_a[r, pl.ds(j * 16, 16)] = jnp.zeros((16,), jnp.float32)
                return 0
            return lax.fori_loop(0, 128 // 16, zcol, 0)
        lax.fori_loop(0, chunk, zrow, 0)
        rbase = sid * _RPT
        nz = _RPT // chunk
        for z in range(nz):
            pltpu.sync_copy(rows_a, accum.at[pl.ds(rbase + z * chunk, chunk)])
        rem = _RPT - nz * chunk
        pltpu.sync_copy(rows_a.at[pl.ds(0, rem)],
                        accum.at[pl.ds(rbase + nz * chunk, rem)])
        plsc.subcore_barrier()

        off0 = cid * _N

        def gather(i, buf, sem):
            idx = sidx.at[pl.ds(i * chunk, chunk)]

            @pl.when(cid == 0)
            def _g0():
                pltpu.make_async_copy(h0_hbm.at[idx], buf, sem).start()

            @pl.when(cid == 1)
            def _g1():
                pltpu.make_async_copy(h1_hbm.at[idx], buf, sem).start()

        def wait_gather(i, buf, sem):
            idx = sidx.at[pl.ds(i * chunk, chunk)]
            pltpu.make_async_copy(h0_hbm.at[idx], buf, sem).wait()

        def scatter(i, buf):
            idx = didx.at[pl.ds(i * chunk, chunk)]
            pltpu.sync_copy(buf, accum.at[idx], add=True)

        nchunk = 25
        for blk in range(nblk):
            if edge_split:
                ebase = cid * (_E // _NC) + sid * ept + blk * eblk
            else:
                ebase = sid * ept + blk * eblk
            pltpu.sync_copy(src_hbm.at[pl.ds(ebase, eblk)], sidx)
            pltpu.sync_copy(dst_hbm.at[pl.ds(ebase, eblk)], didx)

            gather(0, rows_a, sem_a)

            def pair(p, _):
                i = p * 2
                gather(i + 1, rows_b, sem_b)
                wait_gather(i, rows_a, sem_a)
                scatter(i, rows_a)

                @pl.when(i + 2 < nchunk)
                def _next():
                    gather(i + 2, rows_a, sem_a)
                wait_gather(i + 1, rows_b, sem_b)
                scatter(i + 1, rows_b)
                return 0
            lax.fori_loop(0, nchunk // 2, pair, 0)
            wait_gather(nchunk - 1, rows_a, sem_a)
            scatter(nchunk - 1, rows_a)
        plsc.subcore_barrier()

        cbase = sid * 624
        pltpu.sync_copy(accum.at[pl.ds(cbase, 624)],
                        out_hbm.at[pl.ds(off0 + cbase, 624)])

        @pl.when(sid == _NS - 1)
        def _tail():
            pltpu.sync_copy(accum.at[pl.ds(9984, 16)],
                            out_hbm.at[pl.ds(off0 + 9984, 16)])

    return agg


def kernel(features, edge_index, W0, W1, W2):
    src = edge_index[0].astype(jnp.int32)
    dst = edge_index[1].astype(jnp.int32)
    fagg = _make_agg(False)
    eagg = _make_agg(True)
    hp = _mm_first(features, W0)
    a0 = fagg(hp[0], hp[1], src, dst).reshape(_NC, _N, 128)
    hp = _mm_mid(a0, W1)
    a1 = fagg(hp[0], hp[1], src, dst).reshape(_NC, _N, 128)
    h2 = _mm_full(a1, W2)
    a2 = eagg(h2, h2, src, dst).reshape(_NC, _N, 128)
    return _final_tanh_sum(a2)
